# Initial kernel scaffold; baseline (speedup 1.0000x reference)
import jax
import jax.numpy as jnp
from jax.experimental import pallas as pl


def kernel(x, edge_index, edge_attr, batch, g, r, Wl1, bl1, Wr1, br1, We1, att1, bias1, Wl2, bl2, Wr2, br2, We2, att2, bias2, Wg, bg, Wfc, bfc):
    raise NotImplementedError("write your pallas kernel here")


# speedup vs baseline: 8.2522x; 8.2522x over previous
import functools

import jax
import jax.numpy as jnp
from jax import lax
from jax.experimental import pallas as pl
from jax.experimental.pallas import tpu as pltpu
from jax.experimental.pallas import tpu_sc as plsc

F32 = jnp.float32

NN = 10000
NP = 10240
EE = 320000
NB = 64
HID = 128
ED = 16

NC, NS, LANES = 2, 16, 16
NW = NC * NS
K = 64
NCHUNK = EE // K
NJ = (NCHUNK + NW - 1) // NW
ROWS_PT = NP // NS
ACC_W = 144
PRE_W = 32

RB = 1024
GRID_N = NP // RB
EB = 2000
GRID_E = EE // EB

_mesh = plsc.VectorSubcoreMesh(core_axis_name="c", subcore_axis_name="s")

_GDN = lax.GatherDimensionNumbers(
    offset_dims=(), collapsed_slice_dims=(0,), start_index_map=(0,))


def _lane_perm(v, idx):
    return lax.gather(v, idx[:, None], _GDN, (1,),
                      mode=lax.GatherScatterMode.PROMISE_IN_BOUNDS)


def _pre_body(ea_hbm, dst_hbm, part_hbm, dst_v, ea_v, stage, accum_sh, sem):
    c = lax.axis_index("c")
    s = lax.axis_index("s")
    wid = s * NC + c

    zero = jnp.zeros((LANES,), F32)
    one = jnp.ones((LANES,), F32)

    def zrow(k, _):
        stage[k, pl.ds(0, LANES)] = zero
        stage[k, pl.ds(LANES, LANES)] = zero
        return 0
    lax.fori_loop(0, K, zrow, 0)

    def zcopy(j, _):
        pltpu.sync_copy(stage, accum_sh.at[pl.ds(s * ROWS_PT + j * K, K)])
        return 0
    lax.fori_loop(0, ROWS_PT // K, zcopy, 0)

    def orow(k, _):
        stage[k, pl.ds(LANES, LANES)] = one
        return 0
    lax.fori_loop(0, K, orow, 0)
    plsc.subcore_barrier()

    def chunk_body(j, _):
        ci = wid + j * NW

        @pl.when(ci < NCHUNK)
        def _():
            base = ci * K
            pltpu.sync_copy(dst_hbm.at[pl.ds(base, K)], dst_v)
            pltpu.async_copy(ea_hbm.at[pl.ds(base, K)], ea_v, sem).wait()

            def erow(k, _):
                stage[k, pl.ds(0, LANES)] = ea_v[k, :]
                return 0
            lax.fori_loop(0, K, erow, 0)
            pltpu.sync_copy(stage, accum_sh.at[dst_v], add=True)
        return 0
    lax.fori_loop(0, NJ, chunk_body, 0)
    plsc.subcore_barrier()

    def ocopy(j, _):
        off = s * ROWS_PT + j * K
        pltpu.sync_copy(accum_sh.at[pl.ds(off, K)], part_hbm.at[c, pl.ds(off, K)])
        return 0
    lax.fori_loop(0, ROWS_PT // K, ocopy, 0)


@functools.partial(
    pl.kernel,
    out_type=jax.ShapeDtypeStruct((NC, NP, PRE_W), F32),
    mesh=_mesh,
    scratch_types=[
        pltpu.VMEM((K,), jnp.int32),
        pltpu.VMEM((K, ED), F32),
        pltpu.VMEM((K, PRE_W), F32),
        pltpu.VMEM_SHARED((NP, PRE_W), F32),
        pltpu.SemaphoreType.DMA,
    ],
)
def _pre_call(ea_hbm, dst_hbm, part_hbm, dst_v, ea_v, stage, accum_sh, sem):
    _pre_body(ea_hbm, dst_hbm, part_hbm, dst_v, ea_v, stage, accum_sh, sem)


def _edge_body(xl_hbm, xre_hbm, ew_hbm, src_hbm, dst_hbm, att_hbm,
               part_hbm, pden_hbm,
               src_v, dst_v, xl_rows, xr_rows, ew_rows, stage, stage_d,
               att_v, accum_sh, den_sh, sem1, sem2, sem3):
    c = lax.axis_index("c")
    s = lax.axis_index("s")
    wid = s * NC + c

    zero = jnp.zeros((LANES,), F32)

    def zrow(k, _):
        for cc in range(HID // LANES):
            stage[k, pl.ds(cc * LANES, LANES)] = zero
        return 0
    lax.fori_loop(0, K, zrow, 0)

    def zd(i, _):
        stage_d[pl.ds(i * LANES, LANES)] = zero
        return 0
    lax.fori_loop(0, K // LANES, zd, 0)

    def zcopy(j, _):
        pltpu.sync_copy(stage, accum_sh.at[pl.ds(s * ROWS_PT + j * K, K)])
        pltpu.sync_copy(stage_d, den_sh.at[pl.ds(s * ROWS_PT + j * K, K)])
        return 0
    lax.fori_loop(0, ROWS_PT // K, zcopy, 0)
    pltpu.sync_copy(att_hbm, att_v)
    plsc.subcore_barrier()

    def chunk_body(j, _):
        ci = wid + j * NW

        @pl.when(ci < NCHUNK)
        def _():
            base = ci * K
            pltpu.sync_copy(src_hbm.at[pl.ds(base, K)], src_v)
            pltpu.sync_copy(dst_hbm.at[pl.ds(base, K)], dst_v)
            cp1 = pltpu.async_copy(xl_hbm.at[src_v], xl_rows, sem1)
            cp2 = pltpu.async_copy(xre_hbm.at[dst_v], xr_rows, sem2)
            cp3 = pltpu.async_copy(ew_hbm.at[pl.ds(base, K)], ew_rows, sem3)
            cp1.wait()
            cp2.wait()
            cp3.wait()
            attc = [att_v[pl.ds(cc * LANES, LANES)] for cc in range(8)]
            iota = lax.broadcasted_iota(jnp.int32, (LANES,), 0)
            perms = [jnp.bitwise_xor(iota, sh) for sh in (8, 4, 2, 1)]

            def egroup(gg, _):
                avg = jnp.zeros((LANES,), F32)
                for j in range(LANES):
                    k = gg * LANES + j
                    xlc = []
                    acc = jnp.zeros((LANES,), F32)
                    for cc in range(8):
                        xc = xl_rows[k, pl.ds(cc * LANES, LANES)]
                        ev = xc + xr_rows[k, pl.ds(cc * LANES, LANES)] \
                            + ew_rows[k, pl.ds(cc * LANES, LANES)]
                        lr = jnp.maximum(ev, 0.0) + 0.2 * jnp.minimum(ev, 0.0)
                        acc = acc + lr * attc[cc]
                        xlc.append(xc)
                    for p in perms:
                        acc = acc + _lane_perm(acc, p)
                    av = jnp.exp(acc)
                    for cc in range(8):
                        stage[k, pl.ds(cc * LANES, LANES)] = av * xlc[cc]
                    avg = jnp.where(iota == j, av, avg)
                stage_d[pl.ds(gg * LANES, LANES)] = avg
                return 0
            lax.fori_loop(0, K // LANES, egroup, 0)
            pltpu.sync_copy(stage, accum_sh.at[dst_v], add=True)
            pltpu.sync_copy(stage_d, den_sh.at[dst_v], add=True)
        return 0
    lax.fori_loop(0, NJ, chunk_body, 0)
    plsc.subcore_barrier()

    def ocopy(j, _):
        off = s * ROWS_PT + j * K
        pltpu.sync_copy(accum_sh.at[pl.ds(off, K)], part_hbm.at[c, pl.ds(off, K)])
        return 0
    lax.fori_loop(0, ROWS_PT // K, ocopy, 0)
    pltpu.sync_copy(den_sh.at[pl.ds(s * ROWS_PT, ROWS_PT)],
                    pden_hbm.at[c, pl.ds(s * ROWS_PT, ROWS_PT)])


@functools.partial(
    pl.kernel,
    out_type=[jax.ShapeDtypeStruct((NC, NP, HID), F32),
              jax.ShapeDtypeStruct((NC, NP), F32)],
    mesh=_mesh,
    scratch_types=[
        pltpu.VMEM((K,), jnp.int32),
        pltpu.VMEM((K,), jnp.int32),
        pltpu.VMEM((K, HID), F32),
        pltpu.VMEM((K, HID), F32),
        pltpu.VMEM((K, HID), F32),
        pltpu.VMEM((K, HID), F32),
        pltpu.VMEM((K,), F32),
        pltpu.VMEM((HID,), F32),
        pltpu.VMEM_SHARED((NP, HID), F32),
        pltpu.VMEM_SHARED((NP,), F32),
        pltpu.SemaphoreType.DMA,
        pltpu.SemaphoreType.DMA,
        pltpu.SemaphoreType.DMA,
    ],
)
def _edge_call(xl_hbm, xre_hbm, ew_hbm, src_hbm, dst_hbm, att_hbm,
               part_hbm, pden_hbm,
               src_v, dst_v, xl_rows, xr_rows, ew_rows, stage, stage_d,
               att_v, accum_sh, den_sh, sem1, sem2, sem3):
    _edge_body(xl_hbm, xre_hbm, ew_hbm, src_hbm, dst_hbm, att_hbm,
               part_hbm, pden_hbm,
               src_v, dst_v, xl_rows, xr_rows, ew_rows, stage, stage_d,
               att_v, accum_sh, den_sh, sem1, sem2, sem3)


def _node1_body(x_ref, p0_ref, p1_ref, Wl_ref, bl_ref, Wr_ref, br_ref,
                We_ref, attb_ref, xl_ref, xr_ref, scv_ref, mean_ref):
    x = x_ref[...]
    p0 = p0_ref[...]
    p1 = p1_ref[...]
    cnt = jnp.maximum(p0[:, ED:ED + 1] + p1[:, ED:ED + 1], 1.0)
    mean = (p0[:, :ED] + p1[:, :ED]) / cnt
    xl = jnp.dot(x, Wl_ref[...], preferred_element_type=F32) + bl_ref[...]
    xr = jnp.dot(x, Wr_ref[...], preferred_element_type=F32) + br_ref[...]
    se = xl + xr + jnp.dot(mean, We_ref[...], preferred_element_type=F32)
    lr = jnp.where(se > 0, se, 0.2 * se)
    scf = jnp.dot(lr, attb_ref[...], preferred_element_type=F32)
    xl_ref[...] = xl
    xr_ref[...] = xr
    scv_ref[...] = scf[:, :ED]
    mean_ref[...] = mean


def _node1_call(xp, p0, p1, Wl, bl, Wr, br, We, attb):
    return pl.pallas_call(
        _node1_body,
        grid=(GRID_N,),
        in_specs=[
            pl.BlockSpec((RB, HID), lambda i: (i, 0)),
            pl.BlockSpec((RB, PRE_W), lambda i: (i, 0)),
            pl.BlockSpec((RB, PRE_W), lambda i: (i, 0)),
            pl.BlockSpec((HID, HID), lambda i: (0, 0)),
            pl.BlockSpec((1, HID), lambda i: (0, 0)),
            pl.BlockSpec((HID, HID), lambda i: (0, 0)),
            pl.BlockSpec((1, HID), lambda i: (0, 0)),
            pl.BlockSpec((ED, HID), lambda i: (0, 0)),
            pl.BlockSpec((HID, HID), lambda i: (0, 0)),
        ],
        out_specs=[
            pl.BlockSpec((RB, HID), lambda i: (i, 0)),
            pl.BlockSpec((RB, HID), lambda i: (i, 0)),
            pl.BlockSpec((RB, ED), lambda i: (i, 0)),
            pl.BlockSpec((RB, ED), lambda i: (i, 0)),
        ],
        out_shape=[
            jax.ShapeDtypeStruct((NP, HID), F32),
            jax.ShapeDtypeStruct((NP, HID), F32),
            jax.ShapeDtypeStruct((NP, ED), F32),
            jax.ShapeDtypeStruct((NP, ED), F32),
        ],
    )(xp, p0, p1, Wl, bl, Wr, br, We, attb)


def _comb2_body(p0_ref, p1_ref, pd0_ref, pd1_ref, xl1_ref, scv_ref, mean_ref,
                b1_ref, Wl_ref, bl_ref,
                Wr_ref, br_ref, We_ref, attb_ref, xl_ref, xr_ref, scv2_ref):
    p0 = p0_ref[...]
    p1 = p1_ref[...]
    sd = jnp.exp(-scv_ref[:, :1])
    den = (pd0_ref[:, :1] + pd1_ref[:, :1]) * sd + 1.0
    acc = (p0 + p1) * sd + xl1_ref[...]
    z = acc / (den + 1e-16) + b1_ref[...]
    h1 = jnp.where(z > 0, z, jnp.exp(z) - 1.0)
    xl = jnp.dot(h1, Wl_ref[...], preferred_element_type=F32) + bl_ref[...]
    xr = jnp.dot(h1, Wr_ref[...], preferred_element_type=F32) + br_ref[...]
    se = xl + xr + jnp.dot(mean_ref[...], We_ref[...], preferred_element_type=F32)
    lr = jnp.where(se > 0, se, 0.2 * se)
    scf = jnp.dot(lr, attb_ref[...], preferred_element_type=F32)
    xl_ref[...] = xl
    xr_ref[...] = xr
    scv2_ref[...] = scf[:, :ED]


def _comb2_call(p0, p1, pd0, pd1, xl1, scv1, mean, b1, Wl, bl, Wr, br, We, attb):
    return pl.pallas_call(
        _comb2_body,
        grid=(GRID_N,),
        in_specs=[
            pl.BlockSpec((RB, HID), lambda i: (i, 0)),
            pl.BlockSpec((RB, HID), lambda i: (i, 0)),
            pl.BlockSpec((RB, HID), lambda i: (i, 0)),
            pl.BlockSpec((RB, HID), lambda i: (i, 0)),
            pl.BlockSpec((RB, HID), lambda i: (i, 0)),
            pl.BlockSpec((RB, ED), lambda i: (i, 0)),
            pl.BlockSpec((RB, ED), lambda i: (i, 0)),
            pl.BlockSpec((1, HID), lambda i: (0, 0)),
            pl.BlockSpec((HID, HID), lambda i: (0, 0)),
            pl.BlockSpec((1, HID), lambda i: (0, 0)),
            pl.BlockSpec((HID, HID), lambda i: (0, 0)),
            pl.BlockSpec((1, HID), lambda i: (0, 0)),
            pl.BlockSpec((ED, HID), lambda i: (0, 0)),
            pl.BlockSpec((HID, HID), lambda i: (0, 0)),
        ],
        out_specs=[
            pl.BlockSpec((RB, HID), lambda i: (i, 0)),
            pl.BlockSpec((RB, HID), lambda i: (i, 0)),
            pl.BlockSpec((RB, ED), lambda i: (i, 0)),
        ],
        out_shape=[
            jax.ShapeDtypeStruct((NP, HID), F32),
            jax.ShapeDtypeStruct((NP, HID), F32),
            jax.ShapeDtypeStruct((NP, ED), F32),
        ],
    )(p0, p1, pd0, pd1, xl1, scv1, mean, b1, Wl, bl, Wr, br, We, attb)


def _ew_body(ea_ref, We_ref, ew_ref):
    ew_ref[...] = jnp.dot(ea_ref[...], We_ref[...], preferred_element_type=F32)


def _ew_call(ea, We):
    return pl.pallas_call(
        _ew_body,
        grid=(GRID_E,),
        in_specs=[
            pl.BlockSpec((EB, ED), lambda i: (i, 0)),
            pl.BlockSpec((ED, HID), lambda i: (0, 0)),
        ],
        out_specs=pl.BlockSpec((EB, HID), lambda i: (i, 0)),
        out_shape=jax.ShapeDtypeStruct((EE, HID), F32),
    )(ea, We)


def _final_body(p0_ref, p1_ref, pd0_ref, pd1_ref, xl2_ref, scv_ref, b2_ref,
                Wgb_ref, bgb_ref, batch_ref,
                g_ref, r_ref, Wfcb_ref, bfcb_ref, out_ref,
                h2_scr, gsc_scr, pool_scr, den_scr, m_scr):
    t = pl.program_id(0)
    gi = pl.program_id(1)

    @pl.when((t == 0) & (gi == 0))
    def _():
        m_scr[0, 0] = -jnp.inf

    @pl.when(t == 0)
    def _():
        p0 = p0_ref[...]
        p1 = p1_ref[...]
        sd = jnp.exp(-scv_ref[:, :1])
        den = (pd0_ref[:, :1] + pd1_ref[:, :1]) * sd + 1.0
        acc = (p0 + p1) * sd + xl2_ref[...]
        z = acc / (den + 1e-16) + b2_ref[...]
        h2 = jnp.where(z > 0, z, jnp.exp(z) - 1.0)
        gsc = jnp.dot(h2, Wgb_ref[...], preferred_element_type=F32) + bgb_ref[...]
        h2_scr[pl.ds(gi * RB, RB), :] = h2
        gsc_scr[pl.ds(gi * RB, RB), :] = gsc
        m_scr[0, 0] = jnp.maximum(m_scr[0, 0], jnp.max(gsc))

    @pl.when(t == 1)
    def _():
        @pl.when(gi == 0)
        def _():
            pool_scr[...] = jnp.zeros((NB, HID), F32)
            den_scr[...] = jnp.zeros((NB, HID), F32)

        bv = batch_ref[0, 0, :]
        oh = (lax.broadcasted_iota(jnp.int32, (NB, RB), 0)
              == bv[None, :]).astype(F32)
        h2 = h2_scr[pl.ds(gi * RB, RB), :]
        gsc = gsc_scr[pl.ds(gi * RB, RB), :]
        numf = jnp.exp(gsc - m_scr[0, 0])
        pool_scr[...] += jnp.dot(oh, numf * h2, preferred_element_type=F32)
        den_scr[...] += jnp.dot(oh, numf, preferred_element_type=F32)

        @pl.when(gi == GRID_N - 1)
        def _():
            pooled = pool_scr[...] / (den_scr[...] + 1e-16)
            cat = jnp.concatenate([pooled, g_ref[...], r_ref[...]], axis=1)
            out_ref[...] = jnp.dot(cat, Wfcb_ref[...],
                                   preferred_element_type=F32) + bfcb_ref[...]


def _final_call(p0, p1, pd0, pd1, xl2, scv2, b2, Wgb, bgb, batch3, gm, rm,
                Wfcb, bfcb):
    cat_w = HID + gm.shape[1] + rm.shape[1]
    return pl.pallas_call(
        _final_body,
        grid=(2, GRID_N),
        in_specs=[
            pl.BlockSpec((RB, HID), lambda t, i: (i, 0)),
            pl.BlockSpec((RB, HID), lambda t, i: (i, 0)),
            pl.BlockSpec((RB, HID), lambda t, i: (i, 0)),
            pl.BlockSpec((RB, HID), lambda t, i: (i, 0)),
            pl.BlockSpec((RB, HID), lambda t, i: (i, 0)),
            pl.BlockSpec((RB, ED), lambda t, i: (i, 0)),
            pl.BlockSpec((1, HID), lambda t, i: (0, 0)),
            pl.BlockSpec((HID, HID), lambda t, i: (0, 0)),
            pl.BlockSpec((1, HID), lambda t, i: (0, 0)),
            pl.BlockSpec((1, 1, RB), lambda t, i: (i, 0, 0)),
            pl.BlockSpec((NB, 64), lambda t, i: (0, 0)),
            pl.BlockSpec((NB, 64), lambda t, i: (0, 0)),
            pl.BlockSpec((cat_w, HID), lambda t, i: (0, 0)),
            pl.BlockSpec((1, HID), lambda t, i: (0, 0)),
        ],
        out_specs=pl.BlockSpec((NB, HID), lambda t, i: (0, 0)),
        out_shape=jax.ShapeDtypeStruct((NB, HID), F32),
        scratch_shapes=[
            pltpu.VMEM((NP, HID), F32),
            pltpu.VMEM((NP, HID), F32),
            pltpu.VMEM((NB, HID), F32),
            pltpu.VMEM((NB, HID), F32),
            pltpu.SMEM((1, 1), F32),
        ],
    )(p0, p1, pd0, pd1, xl2, scv2, b2, Wgb, bgb, batch3, gm, rm, Wfcb, bfcb)


def kernel(x, edge_index, edge_attr, batch, g, r,
           Wl1, bl1, Wr1, br1, We1, att1, bias1,
           Wl2, bl2, Wr2, br2, We2, att2, bias2,
           Wg, bg, Wfc, bfc):
    src = edge_index[0]
    dst = edge_index[1]
    xp = jnp.pad(x, ((0, NP - NN), (0, 0)))
    batch3 = jnp.pad(batch, (0, NP - NN), constant_values=NB).reshape(GRID_N, 1, RB)

    att1b = jnp.broadcast_to(att1[:, None], (HID, HID))
    att2b = jnp.broadcast_to(att2[:, None], (HID, HID))
    Wgb = jnp.broadcast_to(Wg, (HID, HID))
    bgb = jnp.broadcast_to(bg.reshape(1, 1), (1, HID))
    Wfcb = jnp.broadcast_to(Wfc, (Wfc.shape[0], HID))
    bfcb = jnp.broadcast_to(bfc.reshape(1, 1), (1, HID))

    pre = _pre_call(edge_attr, dst)
    xl1, xr1, scv1, mean = _node1_call(
        xp, pre[0], pre[1], Wl1, bl1.reshape(1, HID), Wr1, br1.reshape(1, HID),
        We1, att1b)
    ew1 = _ew_call(edge_attr, We1)
    part1, pden1 = _edge_call(xl1, xr1, ew1, src, dst, att1)
    pd10 = jnp.broadcast_to(pden1[0][:, None], (NP, HID))
    pd11 = jnp.broadcast_to(pden1[1][:, None], (NP, HID))
    xl2, xr2, scv2 = _comb2_call(
        part1[0], part1[1], pd10, pd11, xl1, scv1, mean,
        bias1.reshape(1, HID), Wl2,
        bl2.reshape(1, HID), Wr2, br2.reshape(1, HID), We2, att2b)
    ew2 = _ew_call(edge_attr, We2)
    part2, pden2 = _edge_call(xl2, xr2, ew2, src, dst, att2)
    pd20 = jnp.broadcast_to(pden2[0][:, None], (NP, HID))
    pd21 = jnp.broadcast_to(pden2[1][:, None], (NP, HID))
    outm = _final_call(part2[0], part2[1], pd20, pd21, xl2, scv2,
                       bias2.reshape(1, HID),
                       Wgb, bgb, batch3, g, r, Wfcb, bfcb)
    return outm[:, 0]


# speedup vs baseline: 11.8444x; 1.4353x over previous
import functools

import jax
import jax.numpy as jnp
from jax import lax
from jax.experimental import pallas as pl
from jax.experimental.pallas import tpu as pltpu
from jax.experimental.pallas import tpu_sc as plsc

F32 = jnp.float32

NN = 10000
NP = 10240
EE = 320000
NB = 64
HID = 128
ED = 16

NC, NS, LANES = 2, 16, 16
NW = NC * NS
K = 32
NCHUNK = EE // K
NJE = 314
ROWS_PT = NP // NS

KP = 128
NCHUNK_P = EE // KP
NJP = 80

RB = 1024
GRID_N = NP // RB
EB = 2000
GRID_E = EE // EB

_mesh = plsc.VectorSubcoreMesh(core_axis_name="c", subcore_axis_name="s")

_GDN = lax.GatherDimensionNumbers(
    offset_dims=(), collapsed_slice_dims=(0,), start_index_map=(0,))


def _lane_perm(v, idx):
    return lax.gather(v, idx[:, None], _GDN, (1,),
                      mode=lax.GatherScatterMode.PROMISE_IN_BOUNDS)


def _pre_body(ea_hbm, dst_hbm, part_hbm, pcnt_hbm,
              dst_v, ea_v, ones_v, zer_v, sum_sh, cnt_sh, seml, sems):
    c = lax.axis_index("c")
    s = lax.axis_index("s")
    wid = s * NC + c

    zero = jnp.zeros((LANES,), F32)
    one = jnp.ones((LANES,), F32)

    def zrow(k, _):
        ea_v[0][k, :] = zero
        return 0
    lax.fori_loop(0, KP, zrow, 0)

    def zvec(i, _):
        ones_v[pl.ds(i * LANES, LANES)] = one
        zer_v[pl.ds(i * LANES, LANES)] = zero
        return 0
    lax.fori_loop(0, KP // LANES, zvec, 0)

    def zcopy(j, _):
        off = s * ROWS_PT + j * KP
        pltpu.sync_copy(ea_v[0], sum_sh.at[pl.ds(off, KP)])
        pltpu.sync_copy(zer_v, cnt_sh.at[pl.ds(off, KP)])
        return 0
    lax.fori_loop(0, ROWS_PT // KP, zcopy, 0)
    plsc.subcore_barrier()

    def valid(j):
        return (wid + j * NW) < NCHUNK_P

    def issue_loads(b, j):
        base = (wid + j * NW) * KP
        pltpu.async_copy(dst_hbm.at[pl.ds(base, KP)], dst_v[b], seml[b])
        pltpu.async_copy(ea_hbm.at[pl.ds(base, KP)], ea_v[b], seml[b])

    def wait_loads(b):
        pltpu.make_async_copy(dst_hbm.at[pl.ds(0, KP)], dst_v[b], seml[b]).wait()
        pltpu.make_async_copy(ea_hbm.at[pl.ds(0, KP)], ea_v[b], seml[b]).wait()

    def issue_scat(b):
        pltpu.async_copy(ea_v[b], sum_sh.at[dst_v[b]], sems[b], add=True)
        pltpu.async_copy(ones_v, cnt_sh.at[dst_v[b]], sems[b], add=True)

    def wait_scat(b):
        pltpu.make_async_copy(ea_v[b], sum_sh.at[dst_v[b]], sems[b]).wait()
        pltpu.make_async_copy(ones_v, cnt_sh.at[dst_v[b]], sems[b]).wait()

    @pl.when(valid(0))
    def _():
        issue_loads(0, 0)

    def pair_body(jj, _):
        for b in (0, 1):
            j2 = 2 * jj + b

            @pl.when(valid(j2))
            def _():
                wait_loads(b)

            @pl.when((j2 >= 1) & valid(j2 - 1))
            def _():
                wait_scat(1 - b)

            @pl.when(valid(j2 + 1))
            def _():
                issue_loads(1 - b, j2 + 1)

            @pl.when(valid(j2))
            def _():
                issue_scat(b)
        return 0
    lax.fori_loop(0, NJP // 2, pair_body, 0)

    @pl.when(valid(NJP - 1))
    def _():
        wait_scat((NJP - 1) % 2)
    plsc.subcore_barrier()

    def ocopy(j, _):
        off = s * ROWS_PT + j * KP
        pltpu.sync_copy(sum_sh.at[pl.ds(off, KP)], part_hbm.at[c, pl.ds(off, KP)])
        pltpu.sync_copy(cnt_sh.at[pl.ds(off, KP)], pcnt_hbm.at[c, pl.ds(off, KP)])
        return 0
    lax.fori_loop(0, ROWS_PT // KP, ocopy, 0)


@functools.partial(
    pl.kernel,
    out_type=[jax.ShapeDtypeStruct((NC, NP, ED), F32),
              jax.ShapeDtypeStruct((NC, NP), F32)],
    mesh=_mesh,
    scratch_types=[
        [pltpu.VMEM((KP,), jnp.int32), pltpu.VMEM((KP,), jnp.int32)],
        [pltpu.VMEM((KP, ED), F32), pltpu.VMEM((KP, ED), F32)],
        pltpu.VMEM((KP,), F32),
        pltpu.VMEM((KP,), F32),
        pltpu.VMEM_SHARED((NP, ED), F32),
        pltpu.VMEM_SHARED((NP,), F32),
        [pltpu.SemaphoreType.DMA, pltpu.SemaphoreType.DMA],
        [pltpu.SemaphoreType.DMA, pltpu.SemaphoreType.DMA],
    ],
)
def _pre_call(ea_hbm, dst_hbm, part_hbm, pcnt_hbm,
              dst_v, ea_v, ones_v, zer_v, sum_sh, cnt_sh, seml, sems):
    _pre_body(ea_hbm, dst_hbm, part_hbm, pcnt_hbm,
              dst_v, ea_v, ones_v, zer_v, sum_sh, cnt_sh, seml, sems)


def _edge_body(xl_hbm, xre_hbm, ew_hbm, src_hbm, dst_hbm, att_hbm,
               part_hbm, pden_hbm,
               src_v, dst_v, xl_rows, xr_rows, ew_rows, stage, stage_d,
               att_v, accum_sh, den_sh, seml, sems):
    c = lax.axis_index("c")
    s = lax.axis_index("s")
    wid = s * NC + c

    zero = jnp.zeros((LANES,), F32)

    def zrow(k, _):
        for cc in range(HID // LANES):
            stage[0][k, pl.ds(cc * LANES, LANES)] = zero
        return 0
    lax.fori_loop(0, K, zrow, 0)

    def zd(i, _):
        stage_d[0][pl.ds(i * LANES, LANES)] = zero
        return 0
    lax.fori_loop(0, K // LANES, zd, 0)

    def zcopy(j, _):
        off = s * ROWS_PT + j * K
        pltpu.sync_copy(stage[0], accum_sh.at[pl.ds(off, K)])
        pltpu.sync_copy(stage_d[0], den_sh.at[pl.ds(off, K)])
        return 0
    lax.fori_loop(0, ROWS_PT // K, zcopy, 0)
    pltpu.sync_copy(att_hbm, att_v)
    plsc.subcore_barrier()

    attc = [att_v[pl.ds(cc * LANES, LANES)] for cc in range(8)]
    iota = lax.broadcasted_iota(jnp.int32, (LANES,), 0)
    perms = [jnp.bitwise_xor(iota, sh) for sh in (8, 4, 2, 1)]

    def valid(j):
        return (wid + j * NW) < NCHUNK

    def issue_loads(b, j):
        base = (wid + j * NW) * K
        pltpu.async_copy(src_hbm.at[pl.ds(base, K)], src_v[b], seml[b])
        pltpu.async_copy(dst_hbm.at[pl.ds(base, K)], dst_v[b], seml[b])
        pltpu.async_copy(ew_hbm.at[pl.ds(base, K)], ew_rows[b], seml[b])

    def wait_idx(b):
        pltpu.make_async_copy(src_hbm.at[pl.ds(0, K)], src_v[b], seml[b]).wait()
        pltpu.make_async_copy(dst_hbm.at[pl.ds(0, K)], dst_v[b], seml[b]).wait()

    def issue_gathers(b):
        pltpu.async_copy(xl_hbm.at[src_v[b]], xl_rows[b], seml[b])
        pltpu.async_copy(xre_hbm.at[dst_v[b]], xr_rows[b], seml[b])

    def wait_rows(b):
        pltpu.make_async_copy(ew_hbm.at[pl.ds(0, K)], ew_rows[b], seml[b]).wait()
        pltpu.make_async_copy(xl_hbm.at[src_v[b]], xl_rows[b], seml[b]).wait()
        pltpu.make_async_copy(xre_hbm.at[dst_v[b]], xr_rows[b], seml[b]).wait()

    def issue_scat(b):
        pltpu.async_copy(stage[b], accum_sh.at[dst_v[b]], sems[b], add=True)
        pltpu.async_copy(stage_d[b], den_sh.at[dst_v[b]], sems[b], add=True)

    def wait_scat(b):
        pltpu.make_async_copy(stage[b], accum_sh.at[dst_v[b]], sems[b]).wait()
        pltpu.make_async_copy(stage_d[b], den_sh.at[dst_v[b]], sems[b]).wait()

    def compute(b):
        def egroup(gg, _):
            avg = jnp.zeros((LANES,), F32)
            for j in range(LANES):
                k = gg * LANES + j
                xlc = []
                acc = jnp.zeros((LANES,), F32)
                for cc in range(8):
                    xc = xl_rows[b][k, pl.ds(cc * LANES, LANES)]
                    ev = xc + xr_rows[b][k, pl.ds(cc * LANES, LANES)] \
                        + ew_rows[b][k, pl.ds(cc * LANES, LANES)]
                    lr = jnp.maximum(ev, 0.0) + 0.2 * jnp.minimum(ev, 0.0)
                    acc = acc + lr * attc[cc]
                    xlc.append(xc)
                for p in perms:
                    acc = acc + _lane_perm(acc, p)
                av = jnp.exp(acc)
                for cc in range(8):
                    stage[b][k, pl.ds(cc * LANES, LANES)] = av * xlc[cc]
                avg = jnp.where(iota == j, av, avg)
            stage_d[b][pl.ds(gg * LANES, LANES)] = avg
            return 0
        lax.fori_loop(0, K // LANES, egroup, 0)

    @pl.when(valid(0))
    def _():
        issue_loads(0, 0)
        wait_idx(0)
        issue_gathers(0)

    def pair_body(jj, _):
        for b in (0, 1):
            j2 = 2 * jj + b

            @pl.when((j2 >= 1) & valid(j2 - 1))
            def _():
                wait_scat(1 - b)

            @pl.when(valid(j2 + 1))
            def _():
                issue_loads(1 - b, j2 + 1)
                wait_idx(1 - b)
                issue_gathers(1 - b)

            @pl.when(valid(j2))
            def _():
                wait_rows(b)
                compute(b)
                issue_scat(b)
        return 0
    lax.fori_loop(0, NJE // 2, pair_body, 0)

    @pl.when(valid(NJE - 1))
    def _():
        wait_scat((NJE - 1) % 2)
    plsc.subcore_barrier()

    def ocopy(j, _):
        off = s * ROWS_PT + j * K
        pltpu.sync_copy(accum_sh.at[pl.ds(off, K)], part_hbm.at[c, pl.ds(off, K)])
        return 0
    lax.fori_loop(0, ROWS_PT // K, ocopy, 0)
    pltpu.sync_copy(den_sh.at[pl.ds(s * ROWS_PT, ROWS_PT)],
                    pden_hbm.at[c, pl.ds(s * ROWS_PT, ROWS_PT)])


@functools.partial(
    pl.kernel,
    out_type=[jax.ShapeDtypeStruct((NC, NP, HID), F32),
              jax.ShapeDtypeStruct((NC, NP), F32)],
    mesh=_mesh,
    scratch_types=[
        [pltpu.VMEM((K,), jnp.int32), pltpu.VMEM((K,), jnp.int32)],
        [pltpu.VMEM((K,), jnp.int32), pltpu.VMEM((K,), jnp.int32)],
        [pltpu.VMEM((K, HID), F32), pltpu.VMEM((K, HID), F32)],
        [pltpu.VMEM((K, HID), F32), pltpu.VMEM((K, HID), F32)],
        [pltpu.VMEM((K, HID), F32), pltpu.VMEM((K, HID), F32)],
        [pltpu.VMEM((K, HID), F32), pltpu.VMEM((K, HID), F32)],
        [pltpu.VMEM((K,), F32), pltpu.VMEM((K,), F32)],
        pltpu.VMEM((HID,), F32),
        pltpu.VMEM_SHARED((NP, HID), F32),
        pltpu.VMEM_SHARED((NP,), F32),
        [pltpu.SemaphoreType.DMA, pltpu.SemaphoreType.DMA],
        [pltpu.SemaphoreType.DMA, pltpu.SemaphoreType.DMA],
    ],
)
def _edge_call(xl_hbm, xre_hbm, ew_hbm, src_hbm, dst_hbm, att_hbm,
               part_hbm, pden_hbm,
               src_v, dst_v, xl_rows, xr_rows, ew_rows, stage, stage_d,
               att_v, accum_sh, den_sh, seml, sems):
    _edge_body(xl_hbm, xre_hbm, ew_hbm, src_hbm, dst_hbm, att_hbm,
               part_hbm, pden_hbm,
               src_v, dst_v, xl_rows, xr_rows, ew_rows, stage, stage_d,
               att_v, accum_sh, den_sh, seml, sems)


def _node1_body(x_ref, p0_ref, p1_ref, c0_ref, c1_ref, Wl_ref, bl_ref,
                Wr_ref, br_ref, We_ref, attb_ref,
                xl_ref, xr_ref, scv_ref, mean_ref):
    x = x_ref[...]
    p0 = p0_ref[...]
    p1 = p1_ref[...]
    cnt = jnp.maximum(c0_ref[:, :1] + c1_ref[:, :1], 1.0)
    mean = (p0 + p1) / cnt
    xl = jnp.dot(x, Wl_ref[...], preferred_element_type=F32) + bl_ref[...]
    xr = jnp.dot(x, Wr_ref[...], preferred_element_type=F32) + br_ref[...]
    se = xl + xr + jnp.dot(mean, We_ref[...], preferred_element_type=F32)
    lr = jnp.where(se > 0, se, 0.2 * se)
    scf = jnp.dot(lr, attb_ref[...], preferred_element_type=F32)
    xl_ref[...] = xl
    xr_ref[...] = xr
    scv_ref[...] = scf[:, :ED]
    mean_ref[...] = mean


def _node1_call(xp, p0, p1, c0, c1, Wl, bl, Wr, br, We, attb):
    return pl.pallas_call(
        _node1_body,
        grid=(GRID_N,),
        in_specs=[
            pl.BlockSpec((RB, HID), lambda i: (i, 0)),
            pl.BlockSpec((RB, ED), lambda i: (i, 0)),
            pl.BlockSpec((RB, ED), lambda i: (i, 0)),
            pl.BlockSpec((RB, ED), lambda i: (i, 0)),
            pl.BlockSpec((RB, ED), lambda i: (i, 0)),
            pl.BlockSpec((HID, HID), lambda i: (0, 0)),
            pl.BlockSpec((1, HID), lambda i: (0, 0)),
            pl.BlockSpec((HID, HID), lambda i: (0, 0)),
            pl.BlockSpec((1, HID), lambda i: (0, 0)),
            pl.BlockSpec((ED, HID), lambda i: (0, 0)),
            pl.BlockSpec((HID, HID), lambda i: (0, 0)),
        ],
        out_specs=[
            pl.BlockSpec((RB, HID), lambda i: (i, 0)),
            pl.BlockSpec((RB, HID), lambda i: (i, 0)),
            pl.BlockSpec((RB, ED), lambda i: (i, 0)),
            pl.BlockSpec((RB, ED), lambda i: (i, 0)),
        ],
        out_shape=[
            jax.ShapeDtypeStruct((NP, HID), F32),
            jax.ShapeDtypeStruct((NP, HID), F32),
            jax.ShapeDtypeStruct((NP, ED), F32),
            jax.ShapeDtypeStruct((NP, ED), F32),
        ],
    )(xp, p0, p1, c0, c1, Wl, bl, Wr, br, We, attb)


def _comb2_body(p0_ref, p1_ref, pd0_ref, pd1_ref, xl1_ref, scv_ref, mean_ref,
                b1_ref, Wl_ref, bl_ref,
                Wr_ref, br_ref, We_ref, attb_ref, xl_ref, xr_ref, scv2_ref):
    p0 = p0_ref[...]
    p1 = p1_ref[...]
    sd = jnp.exp(-scv_ref[:, :1])
    den = (pd0_ref[:, :1] + pd1_ref[:, :1]) * sd + 1.0
    acc = (p0 + p1) * sd + xl1_ref[...]
    z = acc / (den + 1e-16) + b1_ref[...]
    h1 = jnp.where(z > 0, z, jnp.exp(z) - 1.0)
    xl = jnp.dot(h1, Wl_ref[...], preferred_element_type=F32) + bl_ref[...]
    xr = jnp.dot(h1, Wr_ref[...], preferred_element_type=F32) + br_ref[...]
    se = xl + xr + jnp.dot(mean_ref[...], We_ref[...], preferred_element_type=F32)
    lr = jnp.where(se > 0, se, 0.2 * se)
    scf = jnp.dot(lr, attb_ref[...], preferred_element_type=F32)
    xl_ref[...] = xl
    xr_ref[...] = xr
    scv2_ref[...] = scf[:, :ED]


def _comb2_call(p0, p1, pd0, pd1, xl1, scv1, mean, b1, Wl, bl, Wr, br, We, attb):
    return pl.pallas_call(
        _comb2_body,
        grid=(GRID_N,),
        in_specs=[
            pl.BlockSpec((RB, HID), lambda i: (i, 0)),
            pl.BlockSpec((RB, HID), lambda i: (i, 0)),
            pl.BlockSpec((RB, HID), lambda i: (i, 0)),
            pl.BlockSpec((RB, HID), lambda i: (i, 0)),
            pl.BlockSpec((RB, HID), lambda i: (i, 0)),
            pl.BlockSpec((RB, ED), lambda i: (i, 0)),
            pl.BlockSpec((RB, ED), lambda i: (i, 0)),
            pl.BlockSpec((1, HID), lambda i: (0, 0)),
            pl.BlockSpec((HID, HID), lambda i: (0, 0)),
            pl.BlockSpec((1, HID), lambda i: (0, 0)),
            pl.BlockSpec((HID, HID), lambda i: (0, 0)),
            pl.BlockSpec((1, HID), lambda i: (0, 0)),
            pl.BlockSpec((ED, HID), lambda i: (0, 0)),
            pl.BlockSpec((HID, HID), lambda i: (0, 0)),
        ],
        out_specs=[
            pl.BlockSpec((RB, HID), lambda i: (i, 0)),
            pl.BlockSpec((RB, HID), lambda i: (i, 0)),
            pl.BlockSpec((RB, ED), lambda i: (i, 0)),
        ],
        out_shape=[
            jax.ShapeDtypeStruct((NP, HID), F32),
            jax.ShapeDtypeStruct((NP, HID), F32),
            jax.ShapeDtypeStruct((NP, ED), F32),
        ],
    )(p0, p1, pd0, pd1, xl1, scv1, mean, b1, Wl, bl, Wr, br, We, attb)


def _ew_body(ea_ref, We_ref, ew_ref):
    ew_ref[...] = jnp.dot(ea_ref[...], We_ref[...], preferred_element_type=F32)


def _ew_call(ea, We):
    return pl.pallas_call(
        _ew_body,
        grid=(GRID_E,),
        in_specs=[
            pl.BlockSpec((EB, ED), lambda i: (i, 0)),
            pl.BlockSpec((ED, HID), lambda i: (0, 0)),
        ],
        out_specs=pl.BlockSpec((EB, HID), lambda i: (i, 0)),
        out_shape=jax.ShapeDtypeStruct((EE, HID), F32),
    )(ea, We)


def _final_body(p0_ref, p1_ref, pd0_ref, pd1_ref, xl2_ref, scv_ref, b2_ref,
                Wgb_ref, bgb_ref, batch_ref,
                g_ref, r_ref, Wfcb_ref, bfcb_ref, out_ref,
                h2_scr, gsc_scr, pool_scr, den_scr, m_scr):
    t = pl.program_id(0)
    gi = pl.program_id(1)

    @pl.when((t == 0) & (gi == 0))
    def _():
        m_scr[0, 0] = -jnp.inf

    @pl.when(t == 0)
    def _():
        p0 = p0_ref[...]
        p1 = p1_ref[...]
        sd = jnp.exp(-scv_ref[:, :1])
        den = (pd0_ref[:, :1] + pd1_ref[:, :1]) * sd + 1.0
        acc = (p0 + p1) * sd + xl2_ref[...]
        z = acc / (den + 1e-16) + b2_ref[...]
        h2 = jnp.where(z > 0, z, jnp.exp(z) - 1.0)
        gsc = jnp.dot(h2, Wgb_ref[...], preferred_element_type=F32) + bgb_ref[...]
        h2_scr[pl.ds(gi * RB, RB), :] = h2
        gsc_scr[pl.ds(gi * RB, RB), :] = gsc
        m_scr[0, 0] = jnp.maximum(m_scr[0, 0], jnp.max(gsc))

    @pl.when(t == 1)
    def _():
        @pl.when(gi == 0)
        def _():
            pool_scr[...] = jnp.zeros((NB, HID), F32)
            den_scr[...] = jnp.zeros((NB, HID), F32)

        bv = batch_ref[0, 0, :]
        oh = (lax.broadcasted_iota(jnp.int32, (NB, RB), 0)
              == bv[None, :]).astype(F32)
        h2 = h2_scr[pl.ds(gi * RB, RB), :]
        gsc = gsc_scr[pl.ds(gi * RB, RB), :]
        numf = jnp.exp(gsc - m_scr[0, 0])
        pool_scr[...] += jnp.dot(oh, numf * h2, preferred_element_type=F32)
        den_scr[...] += jnp.dot(oh, numf, preferred_element_type=F32)

        @pl.when(gi == GRID_N - 1)
        def _():
            pooled = pool_scr[...] / (den_scr[...] + 1e-16)
            cat = jnp.concatenate([pooled, g_ref[...], r_ref[...]], axis=1)
            out_ref[...] = jnp.dot(cat, Wfcb_ref[...],
                                   preferred_element_type=F32) + bfcb_ref[...]


def _final_call(p0, p1, pd0, pd1, xl2, scv2, b2, Wgb, bgb, batch3, gm, rm,
                Wfcb, bfcb):
    cat_w = HID + gm.shape[1] + rm.shape[1]
    return pl.pallas_call(
        _final_body,
        grid=(2, GRID_N),
        in_specs=[
            pl.BlockSpec((RB, HID), lambda t, i: (i, 0)),
            pl.BlockSpec((RB, HID), lambda t, i: (i, 0)),
            pl.BlockSpec((RB, HID), lambda t, i: (i, 0)),
            pl.BlockSpec((RB, HID), lambda t, i: (i, 0)),
            pl.BlockSpec((RB, HID), lambda t, i: (i, 0)),
            pl.BlockSpec((RB, ED), lambda t, i: (i, 0)),
            pl.BlockSpec((1, HID), lambda t, i: (0, 0)),
            pl.BlockSpec((HID, HID), lambda t, i: (0, 0)),
            pl.BlockSpec((1, HID), lambda t, i: (0, 0)),
            pl.BlockSpec((1, 1, RB), lambda t, i: (i, 0, 0)),
            pl.BlockSpec((NB, 64), lambda t, i: (0, 0)),
            pl.BlockSpec((NB, 64), lambda t, i: (0, 0)),
            pl.BlockSpec((cat_w, HID), lambda t, i: (0, 0)),
            pl.BlockSpec((1, HID), lambda t, i: (0, 0)),
        ],
        out_specs=pl.BlockSpec((NB, HID), lambda t, i: (0, 0)),
        out_shape=jax.ShapeDtypeStruct((NB, HID), F32),
        scratch_shapes=[
            pltpu.VMEM((NP, HID), F32),
            pltpu.VMEM((NP, HID), F32),
            pltpu.VMEM((NB, HID), F32),
            pltpu.VMEM((NB, HID), F32),
            pltpu.SMEM((1, 1), F32),
        ],
    )(p0, p1, pd0, pd1, xl2, scv2, b2, Wgb, bgb, batch3, gm, rm, Wfcb, bfcb)


def kernel(x, edge_index, edge_attr, batch, g, r,
           Wl1, bl1, Wr1, br1, We1, att1, bias1,
           Wl2, bl2, Wr2, br2, We2, att2, bias2,
           Wg, bg, Wfc, bfc):
    src = edge_index[0]
    dst = edge_index[1]
    xp = jnp.pad(x, ((0, NP - NN), (0, 0)))
    batch3 = jnp.pad(batch, (0, NP - NN), constant_values=NB).reshape(GRID_N, 1, RB)

    att1b = jnp.broadcast_to(att1[:, None], (HID, HID))
    att2b = jnp.broadcast_to(att2[:, None], (HID, HID))
    Wgb = jnp.broadcast_to(Wg, (HID, HID))
    bgb = jnp.broadcast_to(bg.reshape(1, 1), (1, HID))
    Wfcb = jnp.broadcast_to(Wfc, (Wfc.shape[0], HID))
    bfcb = jnp.broadcast_to(bfc.reshape(1, 1), (1, HID))

    psum, pcnt = _pre_call(edge_attr, dst)
    c0b = jnp.broadcast_to(pcnt[0][:, None], (NP, ED))
    c1b = jnp.broadcast_to(pcnt[1][:, None], (NP, ED))
    xl1, xr1, scv1, mean = _node1_call(
        xp, psum[0], psum[1], c0b, c1b, Wl1, bl1.reshape(1, HID), Wr1,
        br1.reshape(1, HID), We1, att1b)
    ew1 = _ew_call(edge_attr, We1)
    part1, pden1 = _edge_call(xl1, xr1, ew1, src, dst, att1)
    pd10 = jnp.broadcast_to(pden1[0][:, None], (NP, HID))
    pd11 = jnp.broadcast_to(pden1[1][:, None], (NP, HID))
    xl2, xr2, scv2 = _comb2_call(
        part1[0], part1[1], pd10, pd11, xl1, scv1, mean,
        bias1.reshape(1, HID), Wl2,
        bl2.reshape(1, HID), Wr2, br2.reshape(1, HID), We2, att2b)
    ew2 = _ew_call(edge_attr, We2)
    part2, pden2 = _edge_call(xl2, xr2, ew2, src, dst, att2)
    pd20 = jnp.broadcast_to(pden2[0][:, None], (NP, HID))
    pd21 = jnp.broadcast_to(pden2[1][:, None], (NP, HID))
    outm = _final_call(part2[0], part2[1], pd20, pd21, xl2, scv2,
                       bias2.reshape(1, HID),
                       Wgb, bgb, batch3, g, r, Wfcb, bfcb)
    return outm[:, 0]


# speedup vs baseline: 12.8009x; 1.0808x over previous
import functools

import jax
import jax.numpy as jnp
from jax import lax
from jax.experimental import pallas as pl
from jax.experimental.pallas import tpu as pltpu
from jax.experimental.pallas import tpu_sc as plsc

F32 = jnp.float32

NN = 10000
NP = 10240
EE = 320000
NB = 64
HID = 128
ED = 16

NC, NS, LANES = 2, 16, 16
NW = NC * NS
K = 32
NCHUNK = EE // K
NJE = 314
ROWS_PT = NP // NS

KP = 128
NCHUNK_P = EE // KP
NJP = 80

RB = 1024
GRID_N = NP // RB
EB = 2000
GRID_E = EE // EB

_mesh = plsc.VectorSubcoreMesh(core_axis_name="c", subcore_axis_name="s")

_GDN = lax.GatherDimensionNumbers(
    offset_dims=(), collapsed_slice_dims=(0,), start_index_map=(0,))


def _lane_perm(v, idx):
    return lax.gather(v, idx[:, None], _GDN, (1,),
                      mode=lax.GatherScatterMode.PROMISE_IN_BOUNDS)


def _pre_body(ea_hbm, dst_hbm, part_hbm, pcnt_hbm,
              dst_v, ea_v, ones_v, zer_v, sum_sh, cnt_sh, seml, sems):
    c = lax.axis_index("c")
    s = lax.axis_index("s")
    wid = s * NC + c

    zero = jnp.zeros((LANES,), F32)
    one = jnp.ones((LANES,), F32)

    def zrow(k, _):
        ea_v[0][k, :] = zero
        return 0
    lax.fori_loop(0, KP, zrow, 0)

    def zvec(i, _):
        ones_v[pl.ds(i * LANES, LANES)] = one
        return 0
    lax.fori_loop(0, KP // LANES, zvec, 0)

    def zvec2(i, _):
        zer_v[pl.ds(i * LANES, LANES)] = zero
        return 0
    lax.fori_loop(0, ROWS_PT // LANES, zvec2, 0)

    def zcopy(j, _):
        off = s * ROWS_PT + j * KP
        pltpu.sync_copy(ea_v[0], sum_sh.at[pl.ds(off, KP)])
        return 0
    lax.fori_loop(0, ROWS_PT // KP, zcopy, 0)
    pltpu.sync_copy(zer_v, cnt_sh.at[pl.ds(s * ROWS_PT, ROWS_PT)])
    plsc.subcore_barrier()

    def valid(j):
        return (j >= 0) & ((wid + j * NW) < NCHUNK_P)

    def issue_loads(b, j):
        base = (wid + j * NW) * KP
        pltpu.async_copy(dst_hbm.at[pl.ds(base, KP)], dst_v[b], seml[b])
        pltpu.async_copy(ea_hbm.at[pl.ds(base, KP)], ea_v[b], seml[b])

    def wait_loads(b):
        pltpu.make_async_copy(dst_hbm.at[pl.ds(0, KP)], dst_v[b], seml[b]).wait()
        pltpu.make_async_copy(ea_hbm.at[pl.ds(0, KP)], ea_v[b], seml[b]).wait()

    def issue_scat(b):
        pltpu.async_copy(ea_v[b], sum_sh.at[dst_v[b]], sems[b], add=True)
        pltpu.async_copy(ones_v, cnt_sh.at[dst_v[b]], sems[b], add=True)

    def wait_scat(b):
        pltpu.make_async_copy(ea_v[b], sum_sh.at[dst_v[b]], sems[b]).wait()
        pltpu.make_async_copy(ones_v, cnt_sh.at[dst_v[b]], sems[b]).wait()

    @pl.when(valid(0))
    def _():
        issue_loads(0, 0)

    def pair_body(jj, _):
        for b in (0, 1):
            j2 = 2 * jj + b

            @pl.when(valid(j2))
            def _():
                wait_loads(b)

            @pl.when((j2 >= 1) & valid(j2 - 1))
            def _():
                wait_scat(1 - b)

            @pl.when(valid(j2 + 1))
            def _():
                issue_loads(1 - b, j2 + 1)

            @pl.when(valid(j2))
            def _():
                issue_scat(b)
        return 0
    lax.fori_loop(0, NJP // 2, pair_body, 0)

    @pl.when(valid(NJP - 1))
    def _():
        wait_scat((NJP - 1) % 2)
    plsc.subcore_barrier()

    def ocopy(j, _):
        off = s * ROWS_PT + j * KP
        pltpu.sync_copy(sum_sh.at[pl.ds(off, KP)], part_hbm.at[c, pl.ds(off, KP)])
        return 0
    lax.fori_loop(0, ROWS_PT // KP, ocopy, 0)
    pltpu.sync_copy(cnt_sh.at[pl.ds(s * ROWS_PT, ROWS_PT)],
                    pcnt_hbm.at[c, pl.ds(s * ROWS_PT, ROWS_PT)])


@functools.partial(
    pl.kernel,
    out_type=[jax.ShapeDtypeStruct((NC, NP, ED), F32),
              jax.ShapeDtypeStruct((NC, NP), F32)],
    mesh=_mesh,
    scratch_types=[
        [pltpu.VMEM((KP,), jnp.int32), pltpu.VMEM((KP,), jnp.int32)],
        [pltpu.VMEM((KP, ED), F32), pltpu.VMEM((KP, ED), F32)],
        pltpu.VMEM((KP,), F32),
        pltpu.VMEM((ROWS_PT,), F32),
        pltpu.VMEM_SHARED((NP, ED), F32),
        pltpu.VMEM_SHARED((NP,), F32),
        [pltpu.SemaphoreType.DMA, pltpu.SemaphoreType.DMA],
        [pltpu.SemaphoreType.DMA, pltpu.SemaphoreType.DMA],
    ],
)
def _pre_call(ea_hbm, dst_hbm, part_hbm, pcnt_hbm,
              dst_v, ea_v, ones_v, zer_v, sum_sh, cnt_sh, seml, sems):
    _pre_body(ea_hbm, dst_hbm, part_hbm, pcnt_hbm,
              dst_v, ea_v, ones_v, zer_v, sum_sh, cnt_sh, seml, sems)


def _edge_body(xl_hbm, xre_hbm, ew_hbm, src_hbm, dst_hbm, att_hbm,
               part_hbm, pden_hbm,
               src_v, dst_v, dst_sc, xl_rows, xr_rows, ew_rows, stage,
               stage_d, att_v, zd_buf, accum_sh, den_sh, seml, sems):
    c = lax.axis_index("c")
    s = lax.axis_index("s")
    wid = s * NC + c

    zero = jnp.zeros((LANES,), F32)

    def zrow(k, _):
        for cc in range(HID // LANES):
            stage[0][k, pl.ds(cc * LANES, LANES)] = zero
        return 0
    lax.fori_loop(0, K, zrow, 0)

    def zd(i, _):
        zd_buf[pl.ds(i * LANES, LANES)] = zero
        return 0
    lax.fori_loop(0, ROWS_PT // LANES, zd, 0)

    def zcopy(j, _):
        off = s * ROWS_PT + j * K
        pltpu.sync_copy(stage[0], accum_sh.at[pl.ds(off, K)])
        return 0
    lax.fori_loop(0, ROWS_PT // K, zcopy, 0)
    pltpu.sync_copy(zd_buf, den_sh.at[pl.ds(s * ROWS_PT, ROWS_PT)])
    pltpu.sync_copy(att_hbm, att_v)
    plsc.subcore_barrier()

    attc = [att_v[pl.ds(cc * LANES, LANES)] for cc in range(8)]
    iota = lax.broadcasted_iota(jnp.int32, (LANES,), 0)
    perms = [jnp.bitwise_xor(iota, sh) for sh in (8, 4, 2, 1)]

    def valid(j):
        return (j >= 0) & ((wid + j * NW) < NCHUNK)

    def issue_loads(b, j):
        base = (wid + j * NW) * K
        pltpu.async_copy(src_hbm.at[pl.ds(base, K)], src_v[b], seml[b])
        pltpu.async_copy(dst_hbm.at[pl.ds(base, K)], dst_v[b], seml[b])
        pltpu.async_copy(ew_hbm.at[pl.ds(base, K)], ew_rows[b], seml[b])

    def wait_idx(b):
        pltpu.make_async_copy(src_hbm.at[pl.ds(0, K)], src_v[b], seml[b]).wait()
        pltpu.make_async_copy(dst_hbm.at[pl.ds(0, K)], dst_v[b], seml[b]).wait()

    def issue_gathers(b):
        pltpu.async_copy(xl_hbm.at[src_v[b]], xl_rows[b], seml[b])
        pltpu.async_copy(xre_hbm.at[dst_v[b]], xr_rows[b], seml[b])

    def wait_rows(b):
        pltpu.make_async_copy(ew_hbm.at[pl.ds(0, K)], ew_rows[b], seml[b]).wait()
        pltpu.make_async_copy(xl_hbm.at[src_v[b]], xl_rows[b], seml[b]).wait()
        pltpu.make_async_copy(xre_hbm.at[dst_v[b]], xr_rows[b], seml[b]).wait()

    def issue_scat(b):
        pltpu.async_copy(stage[b], accum_sh.at[dst_sc[b]], sems[b], add=True)
        pltpu.async_copy(stage_d[b], den_sh.at[dst_sc[b]], sems[b], add=True)

    def wait_scat(b):
        pltpu.make_async_copy(stage[b], accum_sh.at[dst_sc[b]], sems[b]).wait()
        pltpu.make_async_copy(stage_d[b], den_sh.at[dst_sc[b]], sems[b]).wait()

    def compute(b):
        def egroup(gg, _):
            avg = jnp.zeros((LANES,), F32)
            for j in range(LANES):
                k = gg * LANES + j
                xlc = []
                acc = jnp.zeros((LANES,), F32)
                for cc in range(8):
                    xc = xl_rows[b][k, pl.ds(cc * LANES, LANES)]
                    ev = xc + xr_rows[b][k, pl.ds(cc * LANES, LANES)] \
                        + ew_rows[b][k, pl.ds(cc * LANES, LANES)]
                    lr = jnp.maximum(ev, 0.0) + 0.2 * jnp.minimum(ev, 0.0)
                    acc = acc + lr * attc[cc]
                    xlc.append(xc)
                for p in perms:
                    acc = acc + _lane_perm(acc, p)
                av = jnp.exp(acc)
                for cc in range(8):
                    stage[b][k, pl.ds(cc * LANES, LANES)] = av * xlc[cc]
                avg = jnp.where(iota == j, av, avg)
            stage_d[b][pl.ds(gg * LANES, LANES)] = avg
            return 0
        lax.fori_loop(0, K // LANES, egroup, 0)
        for i in range(K // LANES):
            dst_sc[b][pl.ds(i * LANES, LANES)] = dst_v[b][pl.ds(i * LANES, LANES)]

    @pl.when(valid(0))
    def _():
        issue_loads(0, 0)
        wait_idx(0)
        issue_gathers(0)

    def pair_body(jj, _):
        for b in (0, 1):
            j2 = 2 * jj + b

            @pl.when((j2 >= 2) & valid(j2 - 2))
            def _():
                wait_scat(b)

            @pl.when(valid(j2 + 1))
            def _():
                issue_loads(1 - b, j2 + 1)
                wait_idx(1 - b)
                issue_gathers(1 - b)

            @pl.when(valid(j2))
            def _():
                wait_rows(b)
                compute(b)
                issue_scat(b)
        return 0
    lax.fori_loop(0, NJE // 2, pair_body, 0)

    @pl.when(valid(NJE - 2))
    def _():
        wait_scat(0)

    @pl.when(valid(NJE - 1))
    def _():
        wait_scat(1)
    plsc.subcore_barrier()

    def ocopy(j, _):
        off = s * ROWS_PT + j * K
        pltpu.sync_copy(accum_sh.at[pl.ds(off, K)], part_hbm.at[c, pl.ds(off, K)])
        return 0
    lax.fori_loop(0, ROWS_PT // K, ocopy, 0)
    pltpu.sync_copy(den_sh.at[pl.ds(s * ROWS_PT, ROWS_PT)],
                    pden_hbm.at[c, pl.ds(s * ROWS_PT, ROWS_PT)])


@functools.partial(
    pl.kernel,
    out_type=[jax.ShapeDtypeStruct((NC, NP, HID), F32),
              jax.ShapeDtypeStruct((NC, NP), F32)],
    mesh=_mesh,
    scratch_types=[
        [pltpu.VMEM((K,), jnp.int32), pltpu.VMEM((K,), jnp.int32)],
        [pltpu.VMEM((K,), jnp.int32), pltpu.VMEM((K,), jnp.int32)],
        [pltpu.VMEM((K,), jnp.int32), pltpu.VMEM((K,), jnp.int32)],
        [pltpu.VMEM((K, HID), F32), pltpu.VMEM((K, HID), F32)],
        [pltpu.VMEM((K, HID), F32), pltpu.VMEM((K, HID), F32)],
        [pltpu.VMEM((K, HID), F32), pltpu.VMEM((K, HID), F32)],
        [pltpu.VMEM((K, HID), F32), pltpu.VMEM((K, HID), F32)],
        [pltpu.VMEM((K,), F32), pltpu.VMEM((K,), F32)],
        pltpu.VMEM((HID,), F32),
        pltpu.VMEM((ROWS_PT,), F32),
        pltpu.VMEM_SHARED((NP, HID), F32),
        pltpu.VMEM_SHARED((NP,), F32),
        [pltpu.SemaphoreType.DMA, pltpu.SemaphoreType.DMA],
        [pltpu.SemaphoreType.DMA, pltpu.SemaphoreType.DMA],
    ],
)
def _edge_call(xl_hbm, xre_hbm, ew_hbm, src_hbm, dst_hbm, att_hbm,
               part_hbm, pden_hbm,
               src_v, dst_v, dst_sc, xl_rows, xr_rows, ew_rows, stage,
               stage_d, att_v, zd_buf, accum_sh, den_sh, seml, sems):
    _edge_body(xl_hbm, xre_hbm, ew_hbm, src_hbm, dst_hbm, att_hbm,
               part_hbm, pden_hbm,
               src_v, dst_v, dst_sc, xl_rows, xr_rows, ew_rows, stage,
               stage_d, att_v, zd_buf, accum_sh, den_sh, seml, sems)


def _node1_body(x_ref, p0_ref, p1_ref, c0_ref, c1_ref, Wl_ref, bl_ref,
                Wr_ref, br_ref, We_ref, attb_ref,
                xl_ref, xr_ref, scv_ref, mean_ref):
    x = x_ref[...]
    p0 = p0_ref[...]
    p1 = p1_ref[...]
    cnt = jnp.maximum(c0_ref[:, :1] + c1_ref[:, :1], 1.0)
    mean = (p0 + p1) / cnt
    xl = jnp.dot(x, Wl_ref[...], preferred_element_type=F32) + bl_ref[...]
    xr = jnp.dot(x, Wr_ref[...], preferred_element_type=F32) + br_ref[...]
    se = xl + xr + jnp.dot(mean, We_ref[...], preferred_element_type=F32)
    lr = jnp.where(se > 0, se, 0.2 * se)
    scf = jnp.dot(lr, attb_ref[...], preferred_element_type=F32)
    xl_ref[...] = xl
    xr_ref[...] = xr
    scv_ref[...] = scf[:, :ED]
    mean_ref[...] = mean


def _node1_call(xp, p0, p1, c0, c1, Wl, bl, Wr, br, We, attb):
    return pl.pallas_call(
        _node1_body,
        grid=(GRID_N,),
        in_specs=[
            pl.BlockSpec((RB, HID), lambda i: (i, 0)),
            pl.BlockSpec((RB, ED), lambda i: (i, 0)),
            pl.BlockSpec((RB, ED), lambda i: (i, 0)),
            pl.BlockSpec((RB, ED), lambda i: (i, 0)),
            pl.BlockSpec((RB, ED), lambda i: (i, 0)),
            pl.BlockSpec((HID, HID), lambda i: (0, 0)),
            pl.BlockSpec((1, HID), lambda i: (0, 0)),
            pl.BlockSpec((HID, HID), lambda i: (0, 0)),
            pl.BlockSpec((1, HID), lambda i: (0, 0)),
            pl.BlockSpec((ED, HID), lambda i: (0, 0)),
            pl.BlockSpec((HID, HID), lambda i: (0, 0)),
        ],
        out_specs=[
            pl.BlockSpec((RB, HID), lambda i: (i, 0)),
            pl.BlockSpec((RB, HID), lambda i: (i, 0)),
            pl.BlockSpec((RB, ED), lambda i: (i, 0)),
            pl.BlockSpec((RB, ED), lambda i: (i, 0)),
        ],
        out_shape=[
            jax.ShapeDtypeStruct((NP, HID), F32),
            jax.ShapeDtypeStruct((NP, HID), F32),
            jax.ShapeDtypeStruct((NP, ED), F32),
            jax.ShapeDtypeStruct((NP, ED), F32),
        ],
    )(xp, p0, p1, c0, c1, Wl, bl, Wr, br, We, attb)


def _comb2_body(p0_ref, p1_ref, pd0_ref, pd1_ref, xl1_ref, scv_ref, mean_ref,
                b1_ref, Wl_ref, bl_ref,
                Wr_ref, br_ref, We_ref, attb_ref, xl_ref, xr_ref, scv2_ref):
    p0 = p0_ref[...]
    p1 = p1_ref[...]
    sd = jnp.exp(-scv_ref[:, :1])
    den = (pd0_ref[:, :1] + pd1_ref[:, :1]) * sd + 1.0
    acc = (p0 + p1) * sd + xl1_ref[...]
    z = acc / (den + 1e-16) + b1_ref[...]
    h1 = jnp.where(z > 0, z, jnp.exp(z) - 1.0)
    xl = jnp.dot(h1, Wl_ref[...], preferred_element_type=F32) + bl_ref[...]
    xr = jnp.dot(h1, Wr_ref[...], preferred_element_type=F32) + br_ref[...]
    se = xl + xr + jnp.dot(mean_ref[...], We_ref[...], preferred_element_type=F32)
    lr = jnp.where(se > 0, se, 0.2 * se)
    scf = jnp.dot(lr, attb_ref[...], preferred_element_type=F32)
    xl_ref[...] = xl
    xr_ref[...] = xr
    scv2_ref[...] = scf[:, :ED]


def _comb2_call(p0, p1, pd0, pd1, xl1, scv1, mean, b1, Wl, bl, Wr, br, We, attb):
    return pl.pallas_call(
        _comb2_body,
        grid=(GRID_N,),
        in_specs=[
            pl.BlockSpec((RB, HID), lambda i: (i, 0)),
            pl.BlockSpec((RB, HID), lambda i: (i, 0)),
            pl.BlockSpec((RB, HID), lambda i: (i, 0)),
            pl.BlockSpec((RB, HID), lambda i: (i, 0)),
            pl.BlockSpec((RB, HID), lambda i: (i, 0)),
            pl.BlockSpec((RB, ED), lambda i: (i, 0)),
            pl.BlockSpec((RB, ED), lambda i: (i, 0)),
            pl.BlockSpec((1, HID), lambda i: (0, 0)),
            pl.BlockSpec((HID, HID), lambda i: (0, 0)),
            pl.BlockSpec((1, HID), lambda i: (0, 0)),
            pl.BlockSpec((HID, HID), lambda i: (0, 0)),
            pl.BlockSpec((1, HID), lambda i: (0, 0)),
            pl.BlockSpec((ED, HID), lambda i: (0, 0)),
            pl.BlockSpec((HID, HID), lambda i: (0, 0)),
        ],
        out_specs=[
            pl.BlockSpec((RB, HID), lambda i: (i, 0)),
            pl.BlockSpec((RB, HID), lambda i: (i, 0)),
            pl.BlockSpec((RB, ED), lambda i: (i, 0)),
        ],
        out_shape=[
            jax.ShapeDtypeStruct((NP, HID), F32),
            jax.ShapeDtypeStruct((NP, HID), F32),
            jax.ShapeDtypeStruct((NP, ED), F32),
        ],
    )(p0, p1, pd0, pd1, xl1, scv1, mean, b1, Wl, bl, Wr, br, We, attb)


def _ew_body(ea_ref, We1_ref, We2_ref, ew1_ref, ew2_ref):
    ea = ea_ref[...]
    ew1_ref[...] = jnp.dot(ea, We1_ref[...], preferred_element_type=F32)
    ew2_ref[...] = jnp.dot(ea, We2_ref[...], preferred_element_type=F32)


def _ew_call(ea, We1, We2):
    return pl.pallas_call(
        _ew_body,
        grid=(GRID_E,),
        in_specs=[
            pl.BlockSpec((EB, ED), lambda i: (i, 0)),
            pl.BlockSpec((ED, HID), lambda i: (0, 0)),
            pl.BlockSpec((ED, HID), lambda i: (0, 0)),
        ],
        out_specs=[
            pl.BlockSpec((EB, HID), lambda i: (i, 0)),
            pl.BlockSpec((EB, HID), lambda i: (i, 0)),
        ],
        out_shape=[
            jax.ShapeDtypeStruct((EE, HID), F32),
            jax.ShapeDtypeStruct((EE, HID), F32),
        ],
    )(ea, We1, We2)


def _final_body(p0_ref, p1_ref, pd0_ref, pd1_ref, xl2_ref, scv_ref, b2_ref,
                Wgb_ref, bgb_ref, batch_ref,
                g_ref, r_ref, Wfcb_ref, bfcb_ref, out_ref,
                h2_scr, gsc_scr, pool_scr, den_scr, m_scr):
    t = pl.program_id(0)
    gi = pl.program_id(1)

    @pl.when((t == 0) & (gi == 0))
    def _():
        m_scr[0, 0] = -jnp.inf

    @pl.when(t == 0)
    def _():
        p0 = p0_ref[...]
        p1 = p1_ref[...]
        sd = jnp.exp(-scv_ref[:, :1])
        den = (pd0_ref[:, :1] + pd1_ref[:, :1]) * sd + 1.0
        acc = (p0 + p1) * sd + xl2_ref[...]
        z = acc / (den + 1e-16) + b2_ref[...]
        h2 = jnp.where(z > 0, z, jnp.exp(z) - 1.0)
        gsc = jnp.dot(h2, Wgb_ref[...], preferred_element_type=F32) + bgb_ref[...]
        h2_scr[pl.ds(gi * RB, RB), :] = h2
        gsc_scr[pl.ds(gi * RB, RB), :] = gsc
        m_scr[0, 0] = jnp.maximum(m_scr[0, 0], jnp.max(gsc))

    @pl.when(t == 1)
    def _():
        @pl.when(gi == 0)
        def _():
            pool_scr[...] = jnp.zeros((NB, HID), F32)
            den_scr[...] = jnp.zeros((NB, HID), F32)

        bv = batch_ref[0, 0, :]
        oh = (lax.broadcasted_iota(jnp.int32, (NB, RB), 0)
              == bv[None, :]).astype(F32)
        h2 = h2_scr[pl.ds(gi * RB, RB), :]
        gsc = gsc_scr[pl.ds(gi * RB, RB), :]
        numf = jnp.exp(gsc - m_scr[0, 0])
        pool_scr[...] += jnp.dot(oh, numf * h2, preferred_element_type=F32)
        den_scr[...] += jnp.dot(oh, numf, preferred_element_type=F32)

        @pl.when(gi == GRID_N - 1)
        def _():
            pooled = pool_scr[...] / (den_scr[...] + 1e-16)
            cat = jnp.concatenate([pooled, g_ref[...], r_ref[...]], axis=1)
            out_ref[...] = jnp.dot(cat, Wfcb_ref[...],
                                   preferred_element_type=F32) + bfcb_ref[...]


def _final_call(p0, p1, pd0, pd1, xl2, scv2, b2, Wgb, bgb, batch3, gm, rm,
                Wfcb, bfcb):
    cat_w = HID + gm.shape[1] + rm.shape[1]
    return pl.pallas_call(
        _final_body,
        grid=(2, GRID_N),
        in_specs=[
            pl.BlockSpec((RB, HID), lambda t, i: (i, 0)),
            pl.BlockSpec((RB, HID), lambda t, i: (i, 0)),
            pl.BlockSpec((RB, HID), lambda t, i: (i, 0)),
            pl.BlockSpec((RB, HID), lambda t, i: (i, 0)),
            pl.BlockSpec((RB, HID), lambda t, i: (i, 0)),
            pl.BlockSpec((RB, ED), lambda t, i: (i, 0)),
            pl.BlockSpec((1, HID), lambda t, i: (0, 0)),
            pl.BlockSpec((HID, HID), lambda t, i: (0, 0)),
            pl.BlockSpec((1, HID), lambda t, i: (0, 0)),
            pl.BlockSpec((1, 1, RB), lambda t, i: (i, 0, 0)),
            pl.BlockSpec((NB, 64), lambda t, i: (0, 0)),
            pl.BlockSpec((NB, 64), lambda t, i: (0, 0)),
            pl.BlockSpec((cat_w, HID), lambda t, i: (0, 0)),
            pl.BlockSpec((1, HID), lambda t, i: (0, 0)),
        ],
        out_specs=pl.BlockSpec((NB, HID), lambda t, i: (0, 0)),
        out_shape=jax.ShapeDtypeStruct((NB, HID), F32),
        scratch_shapes=[
            pltpu.VMEM((NP, HID), F32),
            pltpu.VMEM((NP, HID), F32),
            pltpu.VMEM((NB, HID), F32),
            pltpu.VMEM((NB, HID), F32),
            pltpu.SMEM((1, 1), F32),
        ],
    )(p0, p1, pd0, pd1, xl2, scv2, b2, Wgb, bgb, batch3, gm, rm, Wfcb, bfcb)


def kernel(x, edge_index, edge_attr, batch, g, r,
           Wl1, bl1, Wr1, br1, We1, att1, bias1,
           Wl2, bl2, Wr2, br2, We2, att2, bias2,
           Wg, bg, Wfc, bfc):
    src = edge_index[0]
    dst = edge_index[1]
    xp = jnp.pad(x, ((0, NP - NN), (0, 0)))
    batch3 = jnp.pad(batch, (0, NP - NN), constant_values=NB).reshape(GRID_N, 1, RB)

    att1b = jnp.broadcast_to(att1[:, None], (HID, HID))
    att2b = jnp.broadcast_to(att2[:, None], (HID, HID))
    Wgb = jnp.broadcast_to(Wg, (HID, HID))
    bgb = jnp.broadcast_to(bg.reshape(1, 1), (1, HID))
    Wfcb = jnp.broadcast_to(Wfc, (Wfc.shape[0], HID))
    bfcb = jnp.broadcast_to(bfc.reshape(1, 1), (1, HID))

    psum, pcnt = _pre_call(edge_attr, dst)
    c0b = jnp.broadcast_to(pcnt[0][:, None], (NP, ED))
    c1b = jnp.broadcast_to(pcnt[1][:, None], (NP, ED))
    ew1, ew2 = _ew_call(edge_attr, We1, We2)
    xl1, xr1, scv1, mean = _node1_call(
        xp, psum[0], psum[1], c0b, c1b, Wl1, bl1.reshape(1, HID), Wr1,
        br1.reshape(1, HID), We1, att1b)
    part1, pden1 = _edge_call(xl1, xr1, ew1, src, dst, att1)
    pd10 = jnp.broadcast_to(pden1[0][:, None], (NP, HID))
    pd11 = jnp.broadcast_to(pden1[1][:, None], (NP, HID))
    xl2, xr2, scv2 = _comb2_call(
        part1[0], part1[1], pd10, pd11, xl1, scv1, mean,
        bias1.reshape(1, HID), Wl2,
        bl2.reshape(1, HID), Wr2, br2.reshape(1, HID), We2, att2b)
    part2, pden2 = _edge_call(xl2, xr2, ew2, src, dst, att2)
    pd20 = jnp.broadcast_to(pden2[0][:, None], (NP, HID))
    pd21 = jnp.broadcast_to(pden2[1][:, None], (NP, HID))
    outm = _final_call(part2[0], part2[1], pd20, pd21, xl2, scv2,
                       bias2.reshape(1, HID),
                       Wgb, bgb, batch3, g, r, Wfcb, bfcb)
    return outm[:, 0]
